# baseline (speedup 1.0000x reference)
import functools

import jax
import jax.numpy as jnp
from jax import lax
from jax.experimental import pallas as pl
from jax.experimental.pallas import tpu as pltpu
from jax.experimental.pallas import tpu_sc as plsc

_INFO = plsc.get_sparse_core_info()
_NC = _INFO.num_cores
_NS = _INFO.num_subcores
_NW = _NC * _NS
_LANES = 16

_CHUNK = 49152


def _sc_add(x2, t1, B, W, n_chunks):
    @functools.partial(
        pl.kernel,
        mesh=plsc.VectorSubcoreMesh(core_axis_name="c", subcore_axis_name="s"),
        out_type=jax.ShapeDtypeStruct(x2.shape, jnp.float32),
        scratch_types=[
            pltpu.VMEM((_CHUNK,), jnp.float32),
            pltpu.VMEM((_CHUNK,), jnp.float32),
        ],
    )
    def run(x_hbm, t_hbm, o_hbm, tbuf, xbuf):
        wid = lax.axis_index("s") * _NC + lax.axis_index("c")
        base = wid * W

        def chunk_body(k, carry):
            off = base + k * _CHUNK
            pltpu.sync_copy(t_hbm.at[pl.ds(off, _CHUNK)], tbuf)

            def batch_body(b, c2):
                pltpu.sync_copy(x_hbm.at[b, pl.ds(off, _CHUNK)], xbuf)

                def vec_body(i, c3):
                    sl = pl.ds(i * _LANES, _LANES)
                    plsc.addupdate(xbuf.at[sl], tbuf[sl])
                    return c3

                lax.fori_loop(0, _CHUNK // _LANES, vec_body, 0, unroll=8)
                pltpu.sync_copy(xbuf, o_hbm.at[b, pl.ds(off, _CHUNK)])
                return c2

            lax.fori_loop(0, B, batch_body, 0)
            return carry

        lax.fori_loop(0, n_chunks, chunk_body, 0)

    return run(x2, t1)


def kernel(x, emb_table):
    B, S, D = x.shape
    W = (S // _NW) * D
    n_chunks = W // _CHUNK
    x2 = x.reshape(B, S * D)
    t1 = emb_table.reshape(S * D)
    out = _sc_add(x2, t1, B, W, n_chunks)
    return out.reshape(B, S, D)


# speedup vs baseline: 1.1786x; 1.1786x over previous
import functools

import jax
import jax.numpy as jnp
from jax import lax
from jax.experimental import pallas as pl
from jax.experimental.pallas import tpu as pltpu
from jax.experimental.pallas import tpu_sc as plsc

_INFO = plsc.get_sparse_core_info()
_NC = _INFO.num_cores
_NS = _INFO.num_subcores
_NW = _NC * _NS
_LANES = 16

_CHUNK = 24576


def _sc_add(x2, t1, B, W, n_chunks):
    @functools.partial(
        pl.kernel,
        mesh=plsc.VectorSubcoreMesh(core_axis_name="c", subcore_axis_name="s"),
        out_type=jax.ShapeDtypeStruct(x2.shape, jnp.float32),
        scratch_types=[
            pltpu.VMEM((_CHUNK,), jnp.float32),
            pltpu.VMEM((_CHUNK,), jnp.float32),
            pltpu.VMEM((_CHUNK,), jnp.float32),
            pltpu.VMEM((_CHUNK,), jnp.float32),
            pltpu.SemaphoreType.DMA,
            pltpu.SemaphoreType.DMA,
            pltpu.SemaphoreType.DMA,
            pltpu.SemaphoreType.DMA,
            pltpu.SemaphoreType.DMA,
            pltpu.SemaphoreType.DMA,
        ],
    )
    def run(x_hbm, t_hbm, o_hbm, tbuf0, tbuf1, xbuf0, xbuf1,
            tsem0, tsem1, xsem0, xsem1, osem0, osem1):
        wid = lax.axis_index("s") * _NC + lax.axis_index("c")
        base = wid * W
        tb, tsem = (tbuf0, tbuf1), (tsem0, tsem1)
        xb, xsem = (xbuf0, xbuf1), (xsem0, xsem1)
        osem = (osem0, osem1)

        steps = [(k, b) for k in range(n_chunks) for b in range(B)]
        t_copy = [None, None]
        x_copy = [None, None]
        o_copy = [None, None]

        t_copy[0] = pltpu.async_copy(
            t_hbm.at[pl.ds(base, _CHUNK)], tb[0], tsem[0])
        x_copy[0] = pltpu.async_copy(
            x_hbm.at[0, pl.ds(base, _CHUNK)], xb[0], xsem[0])

        for i, (k, b) in enumerate(steps):
            p = i % 2
            off = base + k * _CHUNK
            if b == 0 and k + 1 < n_chunks:
                t_copy[(k + 1) % 2] = pltpu.async_copy(
                    t_hbm.at[pl.ds(off + _CHUNK, _CHUNK)],
                    tb[(k + 1) % 2], tsem[(k + 1) % 2])
            if i + 1 < len(steps):
                nk, nb = steps[i + 1]
                np_ = (i + 1) % 2
                if o_copy[np_] is not None:
                    o_copy[np_].wait()
                x_copy[np_] = pltpu.async_copy(
                    x_hbm.at[nb, pl.ds(base + nk * _CHUNK, _CHUNK)],
                    xb[np_], xsem[np_])
            if b == 0:
                t_copy[k % 2].wait()
            x_copy[p].wait()

            tref, xref = tb[k % 2], xb[p]

            def vec_body(j, c, tref=tref, xref=xref):
                sl = pl.ds(j * _LANES, _LANES)
                plsc.addupdate(xref.at[sl], tref[sl])
                return c

            lax.fori_loop(0, _CHUNK // _LANES, vec_body, 0, unroll=8)
            o_copy[p] = pltpu.async_copy(
                xref, o_hbm.at[b, pl.ds(off, _CHUNK)], osem[p])

        for p in range(2):
            if o_copy[p] is not None:
                o_copy[p].wait()

    return run(x2, t1)


def kernel(x, emb_table):
    B, S, D = x.shape
    W = (S // _NW) * D
    n_chunks = W // _CHUNK
    x2 = x.reshape(B, S * D)
    t1 = emb_table.reshape(S * D)
    out = _sc_add(x2, t1, B, W, n_chunks)
    return out.reshape(B, S, D)


# speedup vs baseline: 1.2318x; 1.0452x over previous
import functools

import jax
import jax.numpy as jnp
from jax import lax
from jax.experimental import pallas as pl
from jax.experimental.pallas import tpu as pltpu
from jax.experimental.pallas import tpu_sc as plsc

_INFO = plsc.get_sparse_core_info()
_NC = _INFO.num_cores
_NS = _INFO.num_subcores
_NW = _NC * _NS
_LANES = 16

_R = 32


def _sc_add(x, t, B, S, D):
    rows_per_w = S // _NW
    n_chunks = rows_per_w // _R

    @functools.partial(
        pl.kernel,
        mesh=plsc.VectorSubcoreMesh(core_axis_name="c", subcore_axis_name="s"),
        out_type=jax.ShapeDtypeStruct((B, S, D), jnp.float32),
        scratch_types=[
            pltpu.VMEM((_R, D), jnp.float32),
            pltpu.VMEM((_R, D), jnp.float32),
            pltpu.VMEM((_R, D), jnp.float32),
            pltpu.VMEM((_R, D), jnp.float32),
            pltpu.SemaphoreType.DMA,
            pltpu.SemaphoreType.DMA,
            pltpu.SemaphoreType.DMA,
            pltpu.SemaphoreType.DMA,
            pltpu.SemaphoreType.DMA,
            pltpu.SemaphoreType.DMA,
        ],
        compiler_params=pltpu.CompilerParams(use_tc_tiling_on_sc=True),
    )
    def run(x_hbm, t_hbm, o_hbm, tbuf0, tbuf1, xbuf0, xbuf1,
            tsem0, tsem1, xsem0, xsem1, osem0, osem1):
        wid = lax.axis_index("s") * _NC + lax.axis_index("c")
        base = wid * rows_per_w
        tb, tsem = (tbuf0, tbuf1), (tsem0, tsem1)
        xb, xsem = (xbuf0, xbuf1), (xsem0, xsem1)
        osem = (osem0, osem1)

        steps = [(k, b) for k in range(n_chunks) for b in range(B)]
        t_copy = [None, None]
        x_copy = [None, None]
        o_copy = [None, None]

        t_copy[0] = pltpu.async_copy(
            t_hbm.at[pl.ds(base, _R), :], tb[0], tsem[0])
        x_copy[0] = pltpu.async_copy(
            x_hbm.at[0, pl.ds(base, _R), :], xb[0], xsem[0])

        for i, (k, b) in enumerate(steps):
            p = i % 2
            row = base + k * _R
            if b == 0 and k + 1 < n_chunks:
                t_copy[(k + 1) % 2] = pltpu.async_copy(
                    t_hbm.at[pl.ds(row + _R, _R), :],
                    tb[(k + 1) % 2], tsem[(k + 1) % 2])
            if i + 1 < len(steps):
                nk, nb = steps[i + 1]
                np_ = (i + 1) % 2
                if o_copy[np_] is not None:
                    o_copy[np_].wait()
                x_copy[np_] = pltpu.async_copy(
                    x_hbm.at[nb, pl.ds(base + nk * _R, _R), :],
                    xb[np_], xsem[np_])
            if b == 0:
                t_copy[k % 2].wait()
            x_copy[p].wait()

            tref, xref = tb[k % 2], xb[p]

            def row_body(r, c, tref=tref, xref=xref):
                def vec_body(v, c2):
                    sl = pl.ds(v * _LANES, _LANES)
                    plsc.addupdate(xref.at[r, sl], tref[r, sl])
                    return c2

                return lax.fori_loop(0, D // _LANES, vec_body, c, unroll=8)

            lax.fori_loop(0, _R, row_body, 0)
            o_copy[p] = pltpu.async_copy(
                xref, o_hbm.at[b, pl.ds(row, _R), :], osem[p])

        for p in range(2):
            if o_copy[p] is not None:
                o_copy[p].wait()

    return run(x, t)


def kernel(x, emb_table):
    B, S, D = x.shape
    return _sc_add(x, emb_table, B, S, D)


# speedup vs baseline: 2.9133x; 2.3650x over previous
import functools

import jax
import jax.numpy as jnp
from jax import lax
from jax.experimental import pallas as pl
from jax.experimental.pallas import tpu as pltpu
from jax.experimental.pallas import tpu_sc as plsc

try:
    _INFO = plsc.get_sparse_core_info()
    _NC = _INFO.num_cores
    _NS = _INFO.num_subcores
except Exception:
    _NC, _NS = 2, 16
_NW = _NC * _NS
_LANES = 16

_CR = 192


def _add_chunk(xref, tref, CR):
    def row_body(r, c):
        for g in range(128 // _LANES):
            sl = pl.ds(g * _LANES, _LANES)
            plsc.addupdate(xref.at[r, sl], tref[r, sl])
        return c

    lax.fori_loop(0, CR, row_body, 0, unroll=2)


def _sc_add(x3, t2, B, M):
    rows_per_w = M // _NW
    n_chunks = rows_per_w // _CR
    n_pairs = n_chunks // 2

    @functools.partial(
        pl.kernel,
        mesh=plsc.VectorSubcoreMesh(core_axis_name="c", subcore_axis_name="s"),
        out_type=jax.ShapeDtypeStruct((B, M, 128), jnp.float32),
        scratch_types=[
            pltpu.VMEM((_CR, 128), jnp.float32),
            pltpu.VMEM((_CR, 128), jnp.float32),
            pltpu.VMEM((_CR, 128), jnp.float32),
            pltpu.VMEM((_CR, 128), jnp.float32),
            pltpu.SemaphoreType.DMA,
            pltpu.SemaphoreType.DMA,
            pltpu.SemaphoreType.DMA,
            pltpu.SemaphoreType.DMA,
            pltpu.SemaphoreType.DMA,
            pltpu.SemaphoreType.DMA,
        ],
    )
    def run(x_hbm, t_hbm, o_hbm, tbuf0, tbuf1, xbuf0, xbuf1,
            tsem0, tsem1, xsem0, xsem1, osem0, osem1):
        wid = lax.axis_index("s") * _NC + lax.axis_index("c")
        base = wid * rows_per_w
        tb, tsem = (tbuf0, tbuf1), (tsem0, tsem1)
        xb, xsem = (xbuf0, xbuf1), (xsem0, xsem1)
        osem = (osem0, osem1)

        def row0(k):
            return pl.multiple_of(base + k * _CR, 8)

        def t_load(k, kp):
            return pltpu.make_async_copy(
                t_hbm.at[pl.ds(row0(k), _CR), :], tb[kp], tsem[kp])

        def x_load(k, b, p):
            return pltpu.make_async_copy(
                x_hbm.at[b, pl.ds(row0(k), _CR), :], xb[p], xsem[p])

        def o_store(k, b, p):
            return pltpu.make_async_copy(
                xb[p], o_hbm.at[b, pl.ds(row0(k), _CR), :], osem[p])

        t_load(0, 0).start()
        x_load(0, 0, 0).start()

        def pair_body(kk, carry):
            for kp in range(2):
                k = kk * 2 + kp
                for b in range(B):
                    p = b % 2
                    q = (b + 1) % 2
                    if b == 0:
                        if kp == 0:
                            t_load(k + 1, 1).start()
                        else:
                            @pl.when(kk < n_pairs - 1)
                            def _():
                                t_load(k + 1, 0).start()
                        t_load(k, kp).wait()
                    if b == 0:
                        if kp == 1:
                            o_store(k - 1, B - 1, q).wait()
                        else:
                            @pl.when(kk > 0)
                            def _():
                                o_store(k - 1, B - 1, q).wait()
                        x_load(k, 1, q).start()
                    elif b < B - 1:
                        o_store(k, b - 1, q).wait()
                        x_load(k, b + 1, q).start()
                    else:
                        if kp == 0:
                            o_store(k, b - 1, q).wait()
                            x_load(k + 1, 0, q).start()
                        else:
                            @pl.when(kk < n_pairs - 1)
                            def _():
                                o_store(k, b - 1, q).wait()
                                x_load(k + 1, 0, q).start()
                    x_load(k, b, p).wait()
                    _add_chunk(xb[p], tb[kp], _CR)
                    o_store(k, b, p).start()
            return carry

        lax.fori_loop(0, n_pairs, pair_body, 0)

        o_store(n_chunks - 1, B - 2, (B - 2) % 2).wait()
        o_store(n_chunks - 1, B - 1, (B - 1) % 2).wait()

    return run(x3, t2)


def _to_linear_view(a):
    s, d = a.shape[-2], a.shape[-1]
    lead = a.shape[:-2]
    a5 = a.reshape(*lead, s // 8, 8, d // 128, 128)
    perm = tuple(range(len(lead))) + tuple(
        len(lead) + i for i in (0, 2, 1, 3))
    return a5.transpose(perm).reshape(*lead, s * d // 128, 128)


def _from_linear_view(a3, s, d):
    lead = a3.shape[:-2]
    a5 = a3.reshape(*lead, s // 8, d // 128, 8, 128)
    perm = tuple(range(len(lead))) + tuple(
        len(lead) + i for i in (0, 2, 1, 3))
    return a5.transpose(perm).reshape(*lead, s, d)


def kernel(x, emb_table):
    B, S, D = x.shape
    M = S * D // 128
    x3 = _to_linear_view(x)
    t2 = _to_linear_view(emb_table)
    out3 = _sc_add(x3, t2, B, M)
    return _from_linear_view(out3, S, D)
